# baseline (device time: 106566 ns/iter reference)
import jax
import jax.numpy as jnp
from jax import lax
from jax.experimental import pallas as pl
from jax.experimental.pallas import tpu as pltpu

N_DEV = 32
N_STEPS = 5
N_LAYERS = 3


def kernel(x, Win0, Wout0, Win1, Wout1, Win2, Wout2):
    b, d_in = x.shape
    _, h_dim = Win0.shape

    def body(x_ref, win0_ref, wout0_ref, win1_ref, wout1_ref, win2_ref,
             wout2_ref, out_ref, acc_ref, comm_ref, send_sems, recv_sems):
        my = lax.axis_index("i")

        barrier_sem = pltpu.get_barrier_semaphore()
        for s in range(N_STEPS):
            partner = my ^ (1 << s)
            pl.semaphore_signal(
                barrier_sem, inc=1,
                device_id=(partner,), device_id_type=pl.DeviceIdType.MESH,
            )
        pl.semaphore_wait(barrier_sem, N_STEPS)

        x_cur = x_ref[:, :]
        wins = [win0_ref, win1_ref, win2_ref]
        wouts = [wout0_ref, wout1_ref, wout2_ref]
        for l in range(N_LAYERS):
            acc_ref[:, :] = jnp.dot(
                x_cur, wins[l][:, :], preferred_element_type=jnp.float32
            )
            for s in range(N_STEPS):
                idx = l * N_STEPS + s
                partner = my ^ (1 << s)
                rdma = pltpu.make_async_remote_copy(
                    src_ref=acc_ref,
                    dst_ref=comm_ref.at[idx],
                    send_sem=send_sems.at[idx],
                    recv_sem=recv_sems.at[idx],
                    device_id=(partner,),
                    device_id_type=pl.DeviceIdType.MESH,
                )
                rdma.start()
                rdma.wait()
                acc_ref[:, :] = acc_ref[:, :] + comm_ref[idx]
            h = jnp.maximum(acc_ref[:, :], 0.0)
            x_cur = jnp.dot(
                h, wouts[l][:, :], preferred_element_type=jnp.float32
            )
        out_ref[:, :] = x_cur

    return pl.pallas_call(
        body,
        out_shape=jax.ShapeDtypeStruct((b, d_in), jnp.float32),
        in_specs=[pl.BlockSpec(memory_space=pltpu.VMEM)] * 7,
        out_specs=pl.BlockSpec(memory_space=pltpu.VMEM),
        scratch_shapes=[
            pltpu.VMEM((b, h_dim), jnp.float32),
            pltpu.VMEM((N_LAYERS * N_STEPS, b, h_dim), jnp.float32),
            pltpu.SemaphoreType.DMA((N_LAYERS * N_STEPS,)),
            pltpu.SemaphoreType.DMA((N_LAYERS * N_STEPS,)),
        ],
        compiler_params=pltpu.CompilerParams(collective_id=0),
    )(x, Win0, Wout0, Win1, Wout1, Win2, Wout2)


# device time: 74789 ns/iter; 1.4249x vs baseline; 1.4249x over previous
import jax
import jax.numpy as jnp
from jax import lax
from jax.experimental import pallas as pl
from jax.experimental.pallas import tpu as pltpu

N_DEV = 32
N_STEPS = 5
N_LAYERS = 3


def kernel(x, Win0, Wout0, Win1, Wout1, Win2, Wout2):
    b, d_in = x.shape
    _, h_dim = Win0.shape

    def body(x_ref, win0_ref, wout0_ref, win1_ref, wout1_ref, win2_ref,
             wout2_ref, out_ref, send_ref, comm_ref, send_sems, recv_sems):
        my = lax.axis_index("i")

        barrier_sem = pltpu.get_barrier_semaphore()
        for s in range(N_STEPS):
            partner = my ^ (1 << s)
            pl.semaphore_signal(
                barrier_sem, inc=1,
                device_id=(partner,), device_id_type=pl.DeviceIdType.MESH,
            )
        pl.semaphore_wait(barrier_sem, N_STEPS)

        x_cur = x_ref[:, :].astype(jnp.bfloat16)
        wins = [win0_ref, win1_ref, win2_ref]
        wouts = [wout0_ref, wout1_ref, wout2_ref]
        for l in range(N_LAYERS):
            acc = jnp.dot(
                x_cur, wins[l][:, :].astype(jnp.bfloat16),
                preferred_element_type=jnp.float32,
            )
            for s in range(N_STEPS):
                idx = l * N_STEPS + s
                partner = my ^ (1 << s)
                send_ref[:, :] = acc.astype(jnp.bfloat16)
                rdma = pltpu.make_async_remote_copy(
                    src_ref=send_ref,
                    dst_ref=comm_ref.at[idx],
                    send_sem=send_sems.at[idx],
                    recv_sem=recv_sems.at[idx],
                    device_id=(partner,),
                    device_id_type=pl.DeviceIdType.MESH,
                )
                rdma.start()
                rdma.wait()
                acc = acc + comm_ref[idx].astype(jnp.float32)
            h = jnp.maximum(acc, 0.0).astype(jnp.bfloat16)
            x_f32 = jnp.dot(
                h, wouts[l][:, :].astype(jnp.bfloat16),
                preferred_element_type=jnp.float32,
            )
            x_cur = x_f32.astype(jnp.bfloat16)
        out_ref[:, :] = x_f32

    return pl.pallas_call(
        body,
        out_shape=jax.ShapeDtypeStruct((b, d_in), jnp.float32),
        in_specs=[pl.BlockSpec(memory_space=pltpu.VMEM)] * 7,
        out_specs=pl.BlockSpec(memory_space=pltpu.VMEM),
        scratch_shapes=[
            pltpu.VMEM((b, h_dim), jnp.bfloat16),
            pltpu.VMEM((N_LAYERS * N_STEPS, b, h_dim), jnp.bfloat16),
            pltpu.SemaphoreType.DMA((N_LAYERS * N_STEPS,)),
            pltpu.SemaphoreType.DMA((N_LAYERS * N_STEPS,)),
        ],
        compiler_params=pltpu.CompilerParams(collective_id=0),
    )(x, Win0, Wout0, Win1, Wout1, Win2, Wout2)


# device time: 67163 ns/iter; 1.5867x vs baseline; 1.1135x over previous
import jax
import jax.numpy as jnp
from jax import lax
from jax.experimental import pallas as pl
from jax.experimental.pallas import tpu as pltpu

N_DEV = 32
N_STEPS = 5
N_LAYERS = 3


def kernel(x, Win0, Wout0, Win1, Wout1, Win2, Wout2):
    b, d_in = x.shape
    _, h_dim = Win0.shape

    def body(x_ref, win0_ref, wout0_ref, win1_ref, wout1_ref, win2_ref,
             wout2_ref, out_ref, send_ref, comm_ref, send_sems, recv_sems):
        my = lax.axis_index("i")

        barrier_sem = pltpu.get_barrier_semaphore()
        for s in range(N_STEPS):
            partner = my ^ (1 << s)
            pl.semaphore_signal(
                barrier_sem, inc=1,
                device_id=(partner,), device_id_type=pl.DeviceIdType.MESH,
            )
        pl.semaphore_wait(barrier_sem, N_STEPS)

        half = h_dim // 2
        x_cur = x_ref[:, :].astype(jnp.bfloat16)
        wins = [win0_ref, win1_ref, win2_ref]
        wouts = [wout0_ref, wout1_ref, wout2_ref]
        for l in range(N_LAYERS):
            partial = jnp.dot(
                x_cur, wins[l][:, :].astype(jnp.bfloat16),
                preferred_element_type=jnp.float32,
            )
            accs = [partial[:, :half], partial[:, half:]]
            rdmas = {}

            def issue(s, c):
                idx = l * N_STEPS + s
                partner = my ^ (1 << s)
                send_ref[c, :, :] = accs[c].astype(jnp.bfloat16)
                r = pltpu.make_async_remote_copy(
                    src_ref=send_ref.at[c],
                    dst_ref=comm_ref.at[idx, c],
                    send_sem=send_sems.at[idx, c],
                    recv_sem=recv_sems.at[idx, c],
                    device_id=(partner,),
                    device_id_type=pl.DeviceIdType.MESH,
                )
                r.start()
                rdmas[(s, c)] = r

            issue(0, 0)
            issue(0, 1)
            for s in range(N_STEPS):
                idx = l * N_STEPS + s
                for c in (0, 1):
                    rdmas[(s, c)].wait()
                    accs[c] = accs[c] + comm_ref[idx, c].astype(jnp.float32)
                    if s + 1 < N_STEPS:
                        issue(s + 1, c)
            hA = jnp.maximum(accs[0], 0.0).astype(jnp.bfloat16)
            hB = jnp.maximum(accs[1], 0.0).astype(jnp.bfloat16)
            wout_bf = wouts[l][:, :].astype(jnp.bfloat16)
            x_f32 = jnp.dot(
                hA, wout_bf[:half, :], preferred_element_type=jnp.float32
            ) + jnp.dot(
                hB, wout_bf[half:, :], preferred_element_type=jnp.float32
            )
            x_cur = x_f32.astype(jnp.bfloat16)
        out_ref[:, :] = x_f32

    return pl.pallas_call(
        body,
        out_shape=jax.ShapeDtypeStruct((b, d_in), jnp.float32),
        in_specs=[pl.BlockSpec(memory_space=pltpu.VMEM)] * 7,
        out_specs=pl.BlockSpec(memory_space=pltpu.VMEM),
        scratch_shapes=[
            pltpu.VMEM((2, b, h_dim // 2), jnp.bfloat16),
            pltpu.VMEM((N_LAYERS * N_STEPS, 2, b, h_dim // 2), jnp.bfloat16),
            pltpu.SemaphoreType.DMA((N_LAYERS * N_STEPS, 2)),
            pltpu.SemaphoreType.DMA((N_LAYERS * N_STEPS, 2)),
        ],
        compiler_params=pltpu.CompilerParams(collective_id=0),
    )(x, Win0, Wout0, Win1, Wout1, Win2, Wout2)


# device time: 65781 ns/iter; 1.6200x vs baseline; 1.0210x over previous
import jax
import jax.numpy as jnp
from jax import lax
from jax.experimental import pallas as pl
from jax.experimental.pallas import tpu as pltpu

N_DEV = 32
N_STEPS = 5
N_LAYERS = 3
MASKS = (1, 3, 4, 8, 16)


def kernel(x, Win0, Wout0, Win1, Wout1, Win2, Wout2):
    b, d_in = x.shape
    _, h_dim = Win0.shape

    def body(x_ref, win0_ref, wout0_ref, win1_ref, wout1_ref, win2_ref,
             wout2_ref, out_ref, send_ref, comm_ref, send_sems, recv_sems):
        my = lax.axis_index("i")

        barrier_sem = pltpu.get_barrier_semaphore()
        for s in range(N_STEPS):
            partner = my ^ MASKS[s]
            pl.semaphore_signal(
                barrier_sem, inc=1,
                device_id=(partner,), device_id_type=pl.DeviceIdType.MESH,
            )
        pl.semaphore_wait(barrier_sem, N_STEPS)

        half = h_dim // 2
        x_cur = x_ref[:, :].astype(jnp.bfloat16)
        wins = [win0_ref, win1_ref, win2_ref]
        wouts = [wout0_ref, wout1_ref, wout2_ref]
        for l in range(N_LAYERS):
            partial = jnp.dot(
                x_cur, wins[l][:, :].astype(jnp.bfloat16),
                preferred_element_type=jnp.float32,
            )
            send_ref[0, :, :] = partial[:, :half].astype(jnp.bfloat16)
            send_ref[1, :, :] = partial[:, half:].astype(jnp.bfloat16)
            rdmas = {}

            def issue(s, c):
                idx = l * N_STEPS + s
                partner = my ^ MASKS[s]
                r = pltpu.make_async_remote_copy(
                    src_ref=send_ref.at[c],
                    dst_ref=comm_ref.at[idx, c],
                    send_sem=send_sems.at[idx, c],
                    recv_sem=recv_sems.at[idx, c],
                    device_id=(partner,),
                    device_id_type=pl.DeviceIdType.MESH,
                )
                r.start()
                rdmas[(s, c)] = r

            issue(0, 0)
            issue(0, 1)
            for s in range(N_STEPS):
                idx = l * N_STEPS + s
                for c in (0, 1):
                    rdmas[(s, c)].wait()
                    send_ref[c, :, :] = send_ref[c, :, :] + comm_ref[idx, c]
                    if s + 1 < N_STEPS:
                        issue(s + 1, c)
            hA = jnp.maximum(send_ref[0, :, :], 0.0)
            hB = jnp.maximum(send_ref[1, :, :], 0.0)
            wout_bf = wouts[l][:, :].astype(jnp.bfloat16)
            x_f32 = jnp.dot(
                hA, wout_bf[:half, :], preferred_element_type=jnp.float32
            ) + jnp.dot(
                hB, wout_bf[half:, :], preferred_element_type=jnp.float32
            )
            x_cur = x_f32.astype(jnp.bfloat16)
        out_ref[:, :] = x_f32

    return pl.pallas_call(
        body,
        out_shape=jax.ShapeDtypeStruct((b, d_in), jnp.float32),
        in_specs=[pl.BlockSpec(memory_space=pltpu.VMEM)] * 7,
        out_specs=pl.BlockSpec(memory_space=pltpu.VMEM),
        scratch_shapes=[
            pltpu.VMEM((2, b, h_dim // 2), jnp.bfloat16),
            pltpu.VMEM((N_LAYERS * N_STEPS, 2, b, h_dim // 2), jnp.bfloat16),
            pltpu.SemaphoreType.DMA((N_LAYERS * N_STEPS, 2)),
            pltpu.SemaphoreType.DMA((N_LAYERS * N_STEPS, 2)),
        ],
        compiler_params=pltpu.CompilerParams(collective_id=0),
    )(x, Win0, Wout0, Win1, Wout1, Win2, Wout2)
